# baseline (speedup 1.0000x reference)
import functools

import jax
import jax.numpy as jnp
from jax.experimental import pallas as pl

_BLOCK_ROWS = 2000


def _fused_kernel(p_ref, q_ref, ic_ref, w_ref, weu_ref, wei_ref,
                  user_out_ref, item_out_ref):
    f32 = jnp.float32
    user_out_ref[...] = jnp.dot(p_ref[...], weu_ref[...],
                                preferred_element_type=f32)
    wei_top = wei_ref[0:64, :]
    wei_bot = wei_ref[64:128, :]
    w_fold = jnp.dot(w_ref[...], wei_bot, preferred_element_type=f32)
    item_out_ref[...] = (
        jnp.dot(q_ref[...], wei_top, preferred_element_type=f32)
        + jnp.dot(ic_ref[...], w_fold, preferred_element_type=f32)
    )


@jax.jit
def kernel(P, Q, item_content, W, weu, wei):
    n = P.shape[0]
    d = weu.shape[1]
    grid = (n // _BLOCK_ROWS,)
    row_block = lambda i: (i, 0)
    const_block = lambda i: (0, 0)
    user_emb, item_emb = pl.pallas_call(
        _fused_kernel,
        grid=grid,
        in_specs=[
            pl.BlockSpec((_BLOCK_ROWS, P.shape[1]), row_block),
            pl.BlockSpec((_BLOCK_ROWS, Q.shape[1]), row_block),
            pl.BlockSpec((_BLOCK_ROWS, item_content.shape[1]), row_block),
            pl.BlockSpec(W.shape, const_block),
            pl.BlockSpec(weu.shape, const_block),
            pl.BlockSpec(wei.shape, const_block),
        ],
        out_specs=[
            pl.BlockSpec((_BLOCK_ROWS, d), row_block),
            pl.BlockSpec((_BLOCK_ROWS, d), row_block),
        ],
        out_shape=[
            jax.ShapeDtypeStruct((n, d), jnp.float32),
            jax.ShapeDtypeStruct((n, d), jnp.float32),
        ],
    )(P, Q, item_content, W, weu, wei)
    return (user_emb, item_emb)


# speedup vs baseline: 1.0513x; 1.0513x over previous
import functools

import jax
import jax.numpy as jnp
from jax.experimental import pallas as pl

_BLOCK_ROWS = 10000


def _fused_kernel(p_ref, q_ref, ic_ref, w_ref, weu_ref, wei_ref,
                  user_out_ref, item_out_ref):
    f32 = jnp.float32
    user_out_ref[...] = jnp.dot(p_ref[...], weu_ref[...],
                                preferred_element_type=f32)
    wei_top = wei_ref[0:64, :]
    wei_bot = wei_ref[64:128, :]
    w_fold = jnp.dot(w_ref[...], wei_bot, preferred_element_type=f32)
    item_out_ref[...] = (
        jnp.dot(q_ref[...], wei_top, preferred_element_type=f32)
        + jnp.dot(ic_ref[...], w_fold, preferred_element_type=f32)
    )


@jax.jit
def kernel(P, Q, item_content, W, weu, wei):
    n = P.shape[0]
    d = weu.shape[1]
    grid = (n // _BLOCK_ROWS,)
    row_block = lambda i: (i, 0)
    const_block = lambda i: (0, 0)
    user_emb, item_emb = pl.pallas_call(
        _fused_kernel,
        grid=grid,
        in_specs=[
            pl.BlockSpec((_BLOCK_ROWS, P.shape[1]), row_block),
            pl.BlockSpec((_BLOCK_ROWS, Q.shape[1]), row_block),
            pl.BlockSpec((_BLOCK_ROWS, item_content.shape[1]), row_block),
            pl.BlockSpec(W.shape, const_block),
            pl.BlockSpec(weu.shape, const_block),
            pl.BlockSpec(wei.shape, const_block),
        ],
        out_specs=[
            pl.BlockSpec((_BLOCK_ROWS, d), row_block),
            pl.BlockSpec((_BLOCK_ROWS, d), row_block),
        ],
        out_shape=[
            jax.ShapeDtypeStruct((n, d), jnp.float32),
            jax.ShapeDtypeStruct((n, d), jnp.float32),
        ],
    )(P, Q, item_content, W, weu, wei)
    return (user_emb, item_emb)


# speedup vs baseline: 1.0552x; 1.0037x over previous
import functools

import jax
import jax.numpy as jnp
from jax.experimental import pallas as pl
from jax.experimental.pallas import tpu as pltpu

_BLOCK_ROWS = 10000


def _fused_kernel(p_ref, q_ref, ic_ref, w_ref, weu_ref, wei_ref,
                  user_out_ref, item_out_ref):
    f32 = jnp.float32
    user_out_ref[...] = jnp.dot(p_ref[...], weu_ref[...],
                                preferred_element_type=f32)
    wei_top = wei_ref[0:64, :]
    wei_bot = wei_ref[64:128, :]
    w_fold = jnp.dot(w_ref[...], wei_bot, preferred_element_type=f32)
    item_out_ref[...] = (
        jnp.dot(q_ref[...], wei_top, preferred_element_type=f32)
        + jnp.dot(ic_ref[...], w_fold, preferred_element_type=f32)
    )


@jax.jit
def kernel(P, Q, item_content, W, weu, wei):
    n = P.shape[0]
    d = weu.shape[1]
    grid = (n // _BLOCK_ROWS,)
    row_block = lambda i: (i, 0)
    const_block = lambda i: (0, 0)
    user_emb, item_emb = pl.pallas_call(
        _fused_kernel,
        grid=grid,
        in_specs=[
            pl.BlockSpec((_BLOCK_ROWS, P.shape[1]), row_block),
            pl.BlockSpec((_BLOCK_ROWS, Q.shape[1]), row_block),
            pl.BlockSpec((_BLOCK_ROWS, item_content.shape[1]), row_block),
            pl.BlockSpec(W.shape, const_block),
            pl.BlockSpec(weu.shape, const_block),
            pl.BlockSpec(wei.shape, const_block),
        ],
        out_specs=[
            pl.BlockSpec((_BLOCK_ROWS, d), row_block),
            pl.BlockSpec((_BLOCK_ROWS, d), row_block),
        ],
        out_shape=[
            jax.ShapeDtypeStruct((n, d), jnp.float32),
            jax.ShapeDtypeStruct((n, d), jnp.float32),
        ],
        compiler_params=pltpu.CompilerParams(
            dimension_semantics=("parallel",),
        ),
    )(P, Q, item_content, W, weu, wei)
    return (user_emb, item_emb)
